# baseline (speedup 1.0000x reference)
import jax
import jax.numpy as jnp
from jax.experimental import pallas as pl
from jax.experimental.pallas import tpu as pltpu


def _pm_body(x_ref, q_ref, o_ref):
    h = pl.program_id(1)
    w = pl.program_id(2)
    hb = x_ref.shape[1]
    wb = x_ref.shape[2]
    x = x_ref[0]
    q = q_ref[0, 0]
    u = jax.lax.bitcast_convert_type(x, jnp.int32)
    s = (x > q[None, None, :]).astype(jnp.int32)
    u = (u & jnp.int32(-2)) | s
    x2 = jax.lax.bitcast_convert_type(u, jnp.float32)
    m = jnp.max(x2, axis=-1)
    mb = jax.lax.bitcast_convert_type(m, jnp.int32)
    res = (mb & 1) ^ 1
    o_ref[0, 0, pl.ds(w * wb, wb), pl.ds(h * hb, hb)] = res.T


def kernel(input, class_qlims):
    B, H, W, C = input.shape
    HB, WB = 128, 256
    q3 = class_qlims.reshape(B, 1, C)
    grid = (B, H // HB, W // WB)
    return pl.pallas_call(
        _pm_body,
        grid=grid,
        in_specs=[
            pl.BlockSpec((1, HB, WB, C), lambda b, h, w: (b, h, w, 0)),
            pl.BlockSpec((1, 1, C), lambda b, h, w: (b, 0, 0)),
        ],
        out_specs=pl.BlockSpec((1, 1, W, H), lambda b, h, w: (b, 0, 0, 0)),
        out_shape=jax.ShapeDtypeStruct((B, 1, W, H), jnp.int32),
        compiler_params=pltpu.CompilerParams(
            dimension_semantics=("parallel", "arbitrary", "arbitrary"),
        ),
    )(input, q3)
